# Initial kernel scaffold; baseline (speedup 1.0000x reference)
import functools
import jax
import jax.numpy as jnp
from jax import lax
from jax.experimental import pallas as pl
from jax.experimental.pallas import tpu as pltpu
from jax.experimental.pallas import tpu_sc as plsc

NU = 50000
NBD = 20000
NIT = 40000
DH = 32
BATCH = 4096
L = 16
NT = 16
G = 128
NI = 8
EB = G * NI
FCH = 125
UB_E = 1000000
UI_E = 1500000
BI_E = 500000


def _span(E):
    nblk = -(-E // EB)
    out = -(-nblk // NT)
    return out, out * NT * EB


UB_OUT, UB_SPAN = _span(UB_E)
UI_OUT, UI_SPAN = _span(UI_E)
BI_OUT, BI_SPAN = _span(BI_E)
UB_PAD = UB_E + UB_SPAN
UI_PAD = UI_E + UI_SPAN
BI_PAD = BI_SPAN


def _body(ub_r, ub_c, ub_v, ui_r, ui_c, ui_v, bi_r, bi_c, bi_v,
          ub_x0, ui_x0, users, bundles_flat,
          pred,
          ub_x1, ui_x1, ub_m, ui_m, bi_o,
          acc, ebr, ebc, ebv, gidx, sidx, gbuf,
          f0, f1, f2, zbuf,
          uidx, bidx, ug_a, ug_b, bg_a, bg_b, pbuf, sem):
    h = lax.axis_index("c")
    t = lax.axis_index("s")
    third = jnp.float32(1.0 / 3.0)
    zero16 = jnp.zeros((L,), jnp.float32)
    iota = lax.iota(jnp.int32, L)

    @pl.loop(0, G)
    def _(r):
        zbuf[r, pl.ds(0, L)] = zero16
        zbuf[r, pl.ds(L, L)] = zero16

    def spmm_pass(er, ec, ev, e0, E, OUT, src, row_base, col_off, slab,
                  dst, row_off, mean_src=None):
        slab_pt = slab // NT
        r0 = t * slab_pt

        @pl.loop(0, slab_pt // FCH)
        def _(i):
            pltpu.sync_copy(zbuf.at[pl.ds(0, FCH)],
                            acc.at[pl.ds(r0 + i * FCH, FCH)])
        plsc.subcore_barrier()

        @pl.loop(0, OUT)
        def _(o):
            b = o * NT + t
            eoff = e0 + b * EB
            pltpu.sync_copy(er.at[pl.ds(eoff, EB)], ebr)
            pltpu.sync_copy(ec.at[pl.ds(eoff, EB)], ebc)
            pltpu.sync_copy(ev.at[pl.ds(eoff, EB)], ebv)

            @pl.loop(0, NI)
            def _(j):
                @pl.loop(0, G // L)
                def _(k):
                    off = j * G + k * L
                    lim = E - (b * EB + off)
                    m = iota < lim
                    c = ebc[pl.ds(off, L)]
                    r = ebr[pl.ds(off, L)]
                    v = ebv[pl.ds(off, L)]
                    gidx[j, pl.ds(k * L, L)] = jnp.where(m, c + col_off, 0)
                    sidx[j, pl.ds(k * L, L)] = jnp.where(m, r - row_base, 0)
                    ebv[pl.ds(off, L)] = jnp.where(m, v, jnp.float32(0.0))

                pltpu.sync_copy(src.at[h].at[gidx.at[j]], gbuf)

                @pl.loop(0, G, unroll=8)
                def _(e):
                    v = ebv[j * G + e]
                    gbuf[e, pl.ds(0, L)] = gbuf[e, pl.ds(0, L)] * v
                    gbuf[e, pl.ds(L, L)] = gbuf[e, pl.ds(L, L)] * v

                pltpu.sync_copy(gbuf, acc.at[sidx.at[j]], add=True)

        plsc.subcore_barrier()

        if mean_src is None:
            pltpu.sync_copy(acc.at[pl.ds(r0, slab_pt)],
                            dst.at[h].at[pl.ds(row_off + r0, slab_pt)])
        else:
            x0, x1 = mean_src

            @pl.loop(0, slab_pt // FCH)
            def _(i):
                r = r0 + i * FCH
                pltpu.sync_copy(acc.at[pl.ds(r, FCH)], f2.at[pl.ds(0, FCH)])
                pltpu.sync_copy(x0.at[h].at[pl.ds(row_off + r, FCH)],
                                f0.at[pl.ds(0, FCH)])
                pltpu.sync_copy(x1.at[h].at[pl.ds(row_off + r, FCH)],
                                f1.at[pl.ds(0, FCH)])

                @pl.loop(0, FCH, unroll=5)
                def _(rr):
                    a = (f0[rr, pl.ds(0, L)] + f1[rr, pl.ds(0, L)]
                         + f2[rr, pl.ds(0, L)])
                    f2[rr, pl.ds(0, L)] = a * third
                    bq = (f0[rr, pl.ds(L, L)] + f1[rr, pl.ds(L, L)]
                          + f2[rr, pl.ds(L, L)])
                    f2[rr, pl.ds(L, L)] = bq * third

                pltpu.sync_copy(f2.at[pl.ds(0, FCH)],
                                dst.at[h].at[pl.ds(row_off + r, FCH)])
        plsc.subcore_barrier()

    spmm_pass(ui_r, ui_c, ui_v, 0, UI_E, UI_OUT, ui_x0, 0, 0, NU, ui_x1, 0)
    spmm_pass(ui_r, ui_c, ui_v, UI_E, UI_E, UI_OUT, ui_x0, NU, 0, NIT,
              ui_x1, NU)
    spmm_pass(ui_r, ui_c, ui_v, 0, UI_E, UI_OUT, ui_x1, 0, 0, NU, ui_m, 0,
              mean_src=(ui_x0, ui_x1))
    spmm_pass(ui_r, ui_c, ui_v, UI_E, UI_E, UI_OUT, ui_x1, NU, 0, NIT,
              ui_m, NU, mean_src=(ui_x0, ui_x1))
    spmm_pass(bi_r, bi_c, bi_v, 0, BI_E, BI_OUT, ui_m, 0, NU, NBD, bi_o, 0)
    spmm_pass(ub_r, ub_c, ub_v, 0, UB_E, UB_OUT, ub_x0, 0, 0, NU, ub_x1, 0)
    spmm_pass(ub_r, ub_c, ub_v, UB_E, UB_E, UB_OUT, ub_x0, NU, 0, NBD,
              ub_x1, NU)
    spmm_pass(ub_r, ub_c, ub_v, 0, UB_E, UB_OUT, ub_x1, 0, 0, NU, ub_m, 0,
              mean_src=(ub_x0, ub_x1))
    spmm_pass(ub_r, ub_c, ub_v, UB_E, UB_E, UB_OUT, ub_x1, NU, 0, NBD,
              ub_m, NU, mean_src=(ub_x0, ub_x1))

    nu_pt = BATCH // NT
    nb_pt = nu_pt * 2
    pltpu.sync_copy(users.at[pl.ds(t * nu_pt, nu_pt)], uidx.at[0])
    pltpu.sync_copy(bundles_flat.at[pl.ds(t * nb_pt, nb_pt)], bidx.at[0])

    @pl.loop(0, nu_pt // G)
    def _(i):
        pltpu.sync_copy(ub_m.at[h].at[uidx.at[0].at[pl.ds(i * G, G)]],
                        ug_a.at[pl.ds(i * G, G)])
        pltpu.sync_copy(ui_m.at[h].at[uidx.at[0].at[pl.ds(i * G, G)]],
                        ug_b.at[pl.ds(i * G, G)])

    @pl.loop(0, nb_pt // L)
    def _(k):
        bd = bidx[0, pl.ds(k * L, L)]
        bidx[1, pl.ds(k * L, L)] = bd + NU

    @pl.loop(0, nb_pt // G)
    def _(i):
        pltpu.sync_copy(ub_m.at[h].at[bidx.at[1].at[pl.ds(i * G, G)]],
                        bg_a.at[pl.ds(i * G, G)])
        pltpu.sync_copy(bi_o.at[h].at[bidx.at[0].at[pl.ds(i * G, G)]],
                        bg_b.at[pl.ds(i * G, G)])

    @pl.loop(0, nu_pt)
    def _(bq):
        u0 = ug_a[bq, pl.ds(0, L)]
        u1 = ug_a[bq, pl.ds(L, L)]
        u2 = ug_b[bq, pl.ds(0, L)]
        u3 = ug_b[bq, pl.ds(L, L)]

        @pl.loop(0, 2)
        def _(jj):
            s = (u0 * bg_a[2 * bq + jj, pl.ds(0, L)]
                 + u1 * bg_a[2 * bq + jj, pl.ds(L, L)]
                 + u2 * bg_b[2 * bq + jj, pl.ds(0, L)]
                 + u3 * bg_b[2 * bq + jj, pl.ds(L, L)])
            pbuf[bq, jj] = jnp.sum(s)

    pltpu.sync_copy(pbuf, pred.at[h].at[pl.ds(t * nu_pt, nu_pt)])


@jax.jit
def _run(ub_r, ub_c, ub_v, ui_r, ui_c, ui_v, bi_r, bi_c, bi_v,
         ub_x0, ui_x0, users, bundles_flat):
    mesh = plsc.VectorSubcoreMesh(core_axis_name="c", subcore_axis_name="s")
    f32 = jnp.float32
    kfn = pl.kernel(
        _body,
        out_type=jax.ShapeDtypeStruct((2, BATCH, 2), f32),
        mesh=mesh,
        scratch_types=[
            pltpu.HBM((2, NU + NBD, DH), f32),
            pltpu.HBM((2, NU + NIT, DH), f32),
            pltpu.HBM((2, NU + NBD, DH), f32),
            pltpu.HBM((2, NU + NIT, DH), f32),
            pltpu.HBM((2, NBD, DH), f32),
            pltpu.VMEM_SHARED((NU, DH), f32),
            pltpu.VMEM((EB,), jnp.int32),
            pltpu.VMEM((EB,), jnp.int32),
            pltpu.VMEM((EB,), f32),
            pltpu.VMEM((NI, G), jnp.int32),
            pltpu.VMEM((NI, G), jnp.int32),
            pltpu.VMEM((G, DH), f32),
            pltpu.VMEM((G, DH), f32),
            pltpu.VMEM((G, DH), f32),
            pltpu.VMEM((G, DH), f32),
            pltpu.VMEM((G, DH), f32),
            pltpu.VMEM((2, BATCH // NT), jnp.int32),
            pltpu.VMEM((2, 2 * BATCH // NT), jnp.int32),
            pltpu.VMEM((BATCH // NT, DH), f32),
            pltpu.VMEM((BATCH // NT, DH), f32),
            pltpu.VMEM((2 * BATCH // NT, DH), f32),
            pltpu.VMEM((2 * BATCH // NT, DH), f32),
            pltpu.VMEM((BATCH // NT, 2), f32),
            pltpu.SemaphoreType.DMA,
        ],
    )
    return kfn(ub_r, ub_c, ub_v, ui_r, ui_c, ui_v, bi_r, bi_c, bi_v,
               ub_x0, ui_x0, users, bundles_flat)


def _pad(x, n):
    return jnp.concatenate([x, jnp.zeros((n - x.shape[0],), x.dtype)])


def kernel(users_feature, bundles_feature, items_feature, ub_vals, ui_vals,
           bi_vals, ub_rows, ub_cols, ui_rows, ui_cols, bi_rows, bi_cols,
           users, bundles):
    ub = jnp.concatenate([users_feature, bundles_feature], axis=0)
    ui = jnp.concatenate([users_feature, items_feature], axis=0)
    ub_x0 = jnp.stack([ub[:, :DH], ub[:, DH:]], axis=0)
    ui_x0 = jnp.stack([ui[:, :DH], ui[:, DH:]], axis=0)

    pred2 = _run(
        _pad(ub_rows, UB_PAD), _pad(ub_cols, UB_PAD), _pad(ub_vals, UB_PAD),
        _pad(ui_rows, UI_PAD), _pad(ui_cols, UI_PAD), _pad(ui_vals, UI_PAD),
        _pad(bi_rows, BI_PAD), _pad(bi_cols, BI_PAD), _pad(bi_vals, BI_PAD),
        ub_x0, ui_x0, users, bundles.reshape(-1))

    p = pred2[0] + pred2[1]
    bpr = jnp.mean(jax.nn.softplus(p[:, 1] - p[:, 0]))
    return (bpr, jnp.zeros((1,), jnp.float32))


# speedup vs baseline: 4.3933x; 4.3933x over previous
import functools
import jax
import jax.numpy as jnp
from jax import lax
from jax.experimental import pallas as pl
from jax.experimental.pallas import tpu as pltpu
from jax.experimental.pallas import tpu_sc as plsc

NU = 50000
NBD = 20000
NIT = 40000
NU_P = 51200
NBD_P = 20480
NIT_P = 40960
DH = 32
BATCH = 4096
L = 16
NT = 16
G = 128
NI = 8
EB = G * NI
FCH = 64
FB = 32
UB_E = 1000000
UI_E = 1500000
BI_E = 500000


def _span(E):
    nblk = -(-E // EB)
    out = -(-nblk // NT)
    return out, out * NT * EB


UB_OUT, UB_SPAN = _span(UB_E)
UI_OUT, UI_SPAN = _span(UI_E)
BI_OUT, BI_SPAN = _span(BI_E)
UB_PAD = UB_E + UB_SPAN
UI_PAD = UI_E + UI_SPAN
BI_PAD = BI_SPAN


def _body(ub_r, ub_c, ub_v, ui_r, ui_c, ui_v, bi_r, bi_c, bi_v,
          ub_x0, ui_x0, users, bundles_flat, zrows,
          pred,
          ub_x1, ui_x1, ub_m, ui_m, bi_o,
          acc, ebr, ebc, ebv, gidx, sidx, gbuf,
          f0, f1, f2,
          uidx, bidx, ug_a, ug_b, bg_a, bg_b, pbuf, sem):
    h = lax.axis_index("c")
    t = lax.axis_index("s")
    third = jnp.float32(1.0 / 3.0)
    zero16 = jnp.zeros((L,), jnp.float32)
    iota = lax.iota(jnp.int32, L)

    def spmm_pass(er, ec, ev, e0, E, OUT, src, row_base, col_off, slab,
                  dst, row_off, mean_src=None):
        slab_pt = slab // NT
        r0 = t * slab_pt

        pltpu.sync_copy(zrows.at[pl.ds(0, slab_pt)],
                        acc.at[pl.ds(r0, slab_pt)])
        plsc.subcore_barrier()

        @pl.loop(0, OUT)
        def _(o):
            b = o * NT + t
            eoff = e0 + b * EB
            pltpu.sync_copy(er.at[pl.ds(eoff, EB)], ebr)
            pltpu.sync_copy(ec.at[pl.ds(eoff, EB)], ebc)
            pltpu.sync_copy(ev.at[pl.ds(eoff, EB)], ebv)

            @pl.loop(0, NI)
            def _(j):
                @pl.loop(0, G // L)
                def _(k):
                    off = j * G + k * L
                    lim = E - (b * EB + off)
                    m = iota < lim
                    c = ebc[pl.ds(off, L)]
                    r = ebr[pl.ds(off, L)]
                    v = ebv[pl.ds(off, L)]
                    gidx[j, pl.ds(k * L, L)] = jnp.where(m, c + col_off, 0)
                    sidx[j, pl.ds(k * L, L)] = jnp.where(m, r - row_base, 0)
                    ebv[pl.ds(off, L)] = jnp.where(m, v, jnp.float32(0.0))

                pltpu.sync_copy(src.at[h].at[gidx.at[j]], gbuf)

                @pl.loop(0, G // L)
                def _(k):
                    vv = ebv[pl.ds(j * G + k * L, L)]
                    for e2 in range(L):
                        e = k * L + e2
                        v = vv[e2]
                        gbuf[e, pl.ds(0, L)] = gbuf[e, pl.ds(0, L)] * v
                        gbuf[e, pl.ds(L, L)] = gbuf[e, pl.ds(L, L)] * v

                pltpu.sync_copy(gbuf, acc.at[sidx.at[j]], add=True)

        plsc.subcore_barrier()

        if mean_src is None:
            pltpu.sync_copy(acc.at[pl.ds(r0, slab_pt)],
                            dst.at[h].at[pl.ds(row_off + r0, slab_pt)])
        else:
            x0, x1 = mean_src

            @pl.loop(0, slab_pt // FCH)
            def _(i):
                r = r0 + i * FCH
                pltpu.sync_copy(acc.at[pl.ds(r, FCH)], f2.at[pl.ds(0, FCH)])
                pltpu.sync_copy(x0.at[h].at[pl.ds(row_off + r, FCH)],
                                f0.at[pl.ds(0, FCH)])
                pltpu.sync_copy(x1.at[h].at[pl.ds(row_off + r, FCH)],
                                f1.at[pl.ds(0, FCH)])

                @pl.loop(0, FCH, unroll=5)
                def _(rr):
                    a = (f0[rr, pl.ds(0, L)] + f1[rr, pl.ds(0, L)]
                         + f2[rr, pl.ds(0, L)])
                    f2[rr, pl.ds(0, L)] = a * third
                    bq = (f0[rr, pl.ds(L, L)] + f1[rr, pl.ds(L, L)]
                          + f2[rr, pl.ds(L, L)])
                    f2[rr, pl.ds(L, L)] = bq * third

                pltpu.sync_copy(f2.at[pl.ds(0, FCH)],
                                dst.at[h].at[pl.ds(row_off + r, FCH)])
        plsc.subcore_barrier()

    spmm_pass(ui_r, ui_c, ui_v, 0, UI_E, UI_OUT, ui_x0, 0, NU_P - NU,
              NU_P, ui_x1, 0)
    spmm_pass(ui_r, ui_c, ui_v, UI_E, UI_E, UI_OUT, ui_x0, NU, 0, NIT_P,
              ui_x1, NU_P)
    spmm_pass(ui_r, ui_c, ui_v, 0, UI_E, UI_OUT, ui_x1, 0, NU_P - NU,
              NU_P, ui_m, 0, mean_src=(ui_x0, ui_x1))
    spmm_pass(ui_r, ui_c, ui_v, UI_E, UI_E, UI_OUT, ui_x1, NU, 0, NIT_P,
              ui_m, NU_P, mean_src=(ui_x0, ui_x1))
    spmm_pass(bi_r, bi_c, bi_v, 0, BI_E, BI_OUT, ui_m, 0, NU_P, NBD_P,
              bi_o, 0)
    spmm_pass(ub_r, ub_c, ub_v, 0, UB_E, UB_OUT, ub_x0, 0, NU_P - NU,
              NU_P, ub_x1, 0)
    spmm_pass(ub_r, ub_c, ub_v, UB_E, UB_E, UB_OUT, ub_x0, NU, 0, NBD_P,
              ub_x1, NU_P)
    spmm_pass(ub_r, ub_c, ub_v, 0, UB_E, UB_OUT, ub_x1, 0, NU_P - NU,
              NU_P, ub_m, 0, mean_src=(ub_x0, ub_x1))
    spmm_pass(ub_r, ub_c, ub_v, UB_E, UB_E, UB_OUT, ub_x1, NU, 0, NBD_P,
              ub_m, NU_P, mean_src=(ub_x0, ub_x1))

    nu_pt = BATCH // NT
    perms = [iota ^ (1 << p) for p in range(4)]

    def lane_sum(s):
        for pm in perms:
            s = s + s.at[pm].get(mode="promise_in_bounds")
        return s

    @pl.loop(0, nu_pt // FB)
    def _(i):
        u_off = t * nu_pt + i * FB
        pltpu.sync_copy(users.at[pl.ds(u_off, FB)], uidx)
        pltpu.sync_copy(bundles_flat.at[pl.ds(2 * u_off, 2 * FB)],
                        bidx.at[0])

        pltpu.sync_copy(ub_m.at[h].at[uidx], ug_a)
        pltpu.sync_copy(ui_m.at[h].at[uidx], ug_b)

        @pl.loop(0, 2 * FB // L)
        def _(k):
            bd = bidx[0, pl.ds(k * L, L)]
            bidx[1, pl.ds(k * L, L)] = bd + NU_P

        pltpu.sync_copy(ub_m.at[h].at[bidx.at[1]], bg_a)
        pltpu.sync_copy(bi_o.at[h].at[bidx.at[0]], bg_b)

        @pl.loop(0, 2 * FB // L)
        def _(g):
            predv = zero16
            for e2 in range(L):
                bq = g * (L // 2) + e2 // 2
                jj = e2 % 2
                u0 = ug_a[bq, pl.ds(0, L)]
                u1 = ug_a[bq, pl.ds(L, L)]
                u2 = ug_b[bq, pl.ds(0, L)]
                u3 = ug_b[bq, pl.ds(L, L)]
                s = (u0 * bg_a[2 * bq + jj, pl.ds(0, L)]
                     + u1 * bg_a[2 * bq + jj, pl.ds(L, L)]
                     + u2 * bg_b[2 * bq + jj, pl.ds(0, L)]
                     + u3 * bg_b[2 * bq + jj, pl.ds(L, L)])
                predv = jnp.where(iota == e2, lane_sum(s), predv)
            pbuf[pl.ds(g * L, L)] = predv

        pltpu.sync_copy(pbuf, pred.at[h].at[pl.ds(2 * u_off, 2 * FB)])


@jax.jit
def _run(ub_r, ub_c, ub_v, ui_r, ui_c, ui_v, bi_r, bi_c, bi_v,
         ub_x0, ui_x0, users, bundles_flat):
    mesh = plsc.VectorSubcoreMesh(core_axis_name="c", subcore_axis_name="s")
    f32 = jnp.float32
    kfn = pl.kernel(
        _body,
        out_type=jax.ShapeDtypeStruct((2, BATCH * 2), f32),
        mesh=mesh,
        compiler_params=pltpu.CompilerParams(use_tc_tiling_on_sc=False),
        scratch_types=[
            pltpu.HBM((2, NU_P + NBD_P, DH), f32),
            pltpu.HBM((2, NU_P + NIT_P, DH), f32),
            pltpu.HBM((2, NU_P + NBD_P, DH), f32),
            pltpu.HBM((2, NU_P + NIT_P, DH), f32),
            pltpu.HBM((2, NBD_P, DH), f32),
            pltpu.VMEM_SHARED((NU_P, DH), f32),
            pltpu.VMEM((EB,), jnp.int32),
            pltpu.VMEM((EB,), jnp.int32),
            pltpu.VMEM((EB,), f32),
            pltpu.VMEM((NI, G), jnp.int32),
            pltpu.VMEM((NI, G), jnp.int32),
            pltpu.VMEM((G, DH), f32),
            pltpu.VMEM((FCH, DH), f32),
            pltpu.VMEM((FCH, DH), f32),
            pltpu.VMEM((FCH, DH), f32),
            pltpu.VMEM((FB,), jnp.int32),
            pltpu.VMEM((2, 2 * FB), jnp.int32),
            pltpu.VMEM((FB, DH), f32),
            pltpu.VMEM((FB, DH), f32),
            pltpu.VMEM((2 * FB, DH), f32),
            pltpu.VMEM((2 * FB, DH), f32),
            pltpu.VMEM((2 * FB,), f32),
            pltpu.SemaphoreType.DMA,
        ],
    )
    zrows = jnp.zeros((NU_P // NT, DH), jnp.float32)
    return kfn(ub_r, ub_c, ub_v, ui_r, ui_c, ui_v, bi_r, bi_c, bi_v,
               ub_x0, ui_x0, users, bundles_flat, zrows)


def _pad(x, n):
    return jnp.concatenate([x, jnp.zeros((n - x.shape[0],), x.dtype)])


def kernel(users_feature, bundles_feature, items_feature, ub_vals, ui_vals,
           bi_vals, ub_rows, ub_cols, ui_rows, ui_cols, bi_rows, bi_cols,
           users, bundles):
    def _halves(a):
        return jnp.stack([a[:, :DH], a[:, DH:]], axis=0)

    ub_x0 = jnp.zeros((2, NU_P + NBD_P, DH), jnp.float32)
    ub_x0 = ub_x0.at[:, :NU].set(_halves(users_feature))
    ub_x0 = ub_x0.at[:, NU_P:NU_P + NBD].set(_halves(bundles_feature))
    ui_x0 = jnp.zeros((2, NU_P + NIT_P, DH), jnp.float32)
    ui_x0 = ui_x0.at[:, :NU].set(_halves(users_feature))
    ui_x0 = ui_x0.at[:, NU_P:NU_P + NIT].set(_halves(items_feature))

    pred2 = _run(
        _pad(ub_rows, UB_PAD), _pad(ub_cols, UB_PAD), _pad(ub_vals, UB_PAD),
        _pad(ui_rows, UI_PAD), _pad(ui_cols, UI_PAD), _pad(ui_vals, UI_PAD),
        _pad(bi_rows, BI_PAD), _pad(bi_cols, BI_PAD), _pad(bi_vals, BI_PAD),
        ub_x0, ui_x0, users, bundles.reshape(-1))

    p = (pred2[0] + pred2[1]).reshape(BATCH, 2)
    bpr = jnp.mean(jax.nn.softplus(p[:, 1] - p[:, 0]))
    return (bpr, jnp.zeros((1,), jnp.float32))


# speedup vs baseline: 5.9589x; 1.3564x over previous
import functools
import jax
import jax.numpy as jnp
from jax import lax
from jax.experimental import pallas as pl
from jax.experimental.pallas import tpu as pltpu
from jax.experimental.pallas import tpu_sc as plsc

NU = 50000
NBD = 20000
NIT = 40000
NU_P = 51200
NBD_P = 20480
NIT_P = 40960
DH = 32
BATCH = 4096
L = 16
NT = 16
G = 128
NI = 8
EB = G * NI
FCH = 32
FB = 32
UB_E = 1000000
UI_E = 1500000
BI_E = 500000


def _span(E):
    nblk = -(-E // EB)
    out = -(-nblk // NT)
    return out, out * NT * EB


UB_OUT, UB_SPAN = _span(UB_E)
UI_OUT, UI_SPAN = _span(UI_E)
BI_OUT, BI_SPAN = _span(BI_E)
UB_PAD = UB_E + UB_SPAN
UI_PAD = UI_E + UI_SPAN
BI_PAD = BI_SPAN


def _body(ub_r, ub_c, ub_v, ui_r, ui_c, ui_v, bi_r, bi_c, bi_v,
          ub_x0, ui_x0, users, bundles_flat, zrows,
          pred,
          ub_x1, ui_x1, ub_m, ui_m, bi_o,
          acc, ebr, ebc, ebv, gidx, sidx, gbuf, gbuf2,
          f0, f1, f2,
          uidx, bidx, ug_a, ug_b, bg_a, bg_b, pbuf, sem, sem2):
    h = lax.axis_index("c")
    t = lax.axis_index("s")
    third = jnp.float32(1.0 / 3.0)
    zero16 = jnp.zeros((L,), jnp.float32)
    iota = lax.iota(jnp.int32, L)

    def spmm_pass(er, ec, ev, e0, E, OUT, src, row_base, col_off, slab,
                  dst, row_off, mean_src=None):
        slab_pt = slab // NT
        r0 = t * slab_pt

        pltpu.sync_copy(zrows.at[pl.ds(0, slab_pt)],
                        acc.at[pl.ds(r0, slab_pt)])
        plsc.subcore_barrier()

        def prep(b, j):
            @pl.loop(0, G // L)
            def _(k):
                off = j * G + k * L
                lim = E - (b * EB + off)
                m = iota < lim
                c = ebc[pl.ds(off, L)]
                r = ebr[pl.ds(off, L)]
                v = ebv[pl.ds(off, L)]
                gidx[j, pl.ds(k * L, L)] = jnp.where(m, c + col_off, 0)
                sidx[j, pl.ds(k * L, L)] = jnp.where(m, r - row_base, 0)
                ebv[pl.ds(off, L)] = jnp.where(m, v, jnp.float32(0.0))

        def fire(j, buf, sm):
            pltpu.async_copy(src.at[h].at[gidx.at[j]], buf, sm)

        def drain(j, buf, sm):
            pltpu.make_async_copy(src.at[h].at[gidx.at[j]], buf, sm).wait()

        def scale_scatter(j, buf):
            @pl.loop(0, G // L)
            def _(k):
                vv = ebv[pl.ds(j * G + k * L, L)]
                for e2 in range(L):
                    e = k * L + e2
                    v = vv[e2]
                    buf[e, pl.ds(0, L)] = buf[e, pl.ds(0, L)] * v
                    buf[e, pl.ds(L, L)] = buf[e, pl.ds(L, L)] * v

            pltpu.sync_copy(buf, acc.at[sidx.at[j]], add=True)

        @pl.loop(0, OUT)
        def _(o):
            b = o * NT + t
            eoff = e0 + b * EB
            pltpu.sync_copy(er.at[pl.ds(eoff, EB)], ebr)
            pltpu.sync_copy(ec.at[pl.ds(eoff, EB)], ebc)
            pltpu.sync_copy(ev.at[pl.ds(eoff, EB)], ebv)

            prep(b, 0)
            fire(0, gbuf, sem)

            @pl.loop(0, NI // 2)
            def _(j2):
                ja = 2 * j2
                jb = ja + 1
                prep(b, jb)
                fire(jb, gbuf2, sem2)
                drain(ja, gbuf, sem)
                scale_scatter(ja, gbuf)

                @pl.when(j2 < NI // 2 - 1)
                def _():
                    prep(b, ja + 2)
                    fire(ja + 2, gbuf, sem)

                drain(jb, gbuf2, sem2)
                scale_scatter(jb, gbuf2)

        plsc.subcore_barrier()

        if mean_src is None:
            pltpu.sync_copy(acc.at[pl.ds(r0, slab_pt)],
                            dst.at[h].at[pl.ds(row_off + r0, slab_pt)])
        else:
            x0, x1 = mean_src

            @pl.loop(0, slab_pt // FCH)
            def _(i):
                r = r0 + i * FCH
                pltpu.sync_copy(acc.at[pl.ds(r, FCH)], f2.at[pl.ds(0, FCH)])
                pltpu.sync_copy(x0.at[h].at[pl.ds(row_off + r, FCH)],
                                f0.at[pl.ds(0, FCH)])
                pltpu.sync_copy(x1.at[h].at[pl.ds(row_off + r, FCH)],
                                f1.at[pl.ds(0, FCH)])

                @pl.loop(0, FCH, unroll=5)
                def _(rr):
                    a = (f0[rr, pl.ds(0, L)] + f1[rr, pl.ds(0, L)]
                         + f2[rr, pl.ds(0, L)])
                    f2[rr, pl.ds(0, L)] = a * third
                    bq = (f0[rr, pl.ds(L, L)] + f1[rr, pl.ds(L, L)]
                          + f2[rr, pl.ds(L, L)])
                    f2[rr, pl.ds(L, L)] = bq * third

                pltpu.sync_copy(f2.at[pl.ds(0, FCH)],
                                dst.at[h].at[pl.ds(row_off + r, FCH)])
        plsc.subcore_barrier()

    spmm_pass(ui_r, ui_c, ui_v, 0, UI_E, UI_OUT, ui_x0, 0, NU_P - NU,
              NU_P, ui_x1, 0)
    spmm_pass(ui_r, ui_c, ui_v, UI_E, UI_E, UI_OUT, ui_x0, NU, 0, NIT_P,
              ui_x1, NU_P)
    spmm_pass(ui_r, ui_c, ui_v, 0, UI_E, UI_OUT, ui_x1, 0, NU_P - NU,
              NU_P, ui_m, 0, mean_src=(ui_x0, ui_x1))
    spmm_pass(ui_r, ui_c, ui_v, UI_E, UI_E, UI_OUT, ui_x1, NU, 0, NIT_P,
              ui_m, NU_P, mean_src=(ui_x0, ui_x1))
    spmm_pass(bi_r, bi_c, bi_v, 0, BI_E, BI_OUT, ui_m, 0, NU_P, NBD_P,
              bi_o, 0)
    spmm_pass(ub_r, ub_c, ub_v, 0, UB_E, UB_OUT, ub_x0, 0, NU_P - NU,
              NU_P, ub_x1, 0)
    spmm_pass(ub_r, ub_c, ub_v, UB_E, UB_E, UB_OUT, ub_x0, NU, 0, NBD_P,
              ub_x1, NU_P)
    spmm_pass(ub_r, ub_c, ub_v, 0, UB_E, UB_OUT, ub_x1, 0, NU_P - NU,
              NU_P, ub_m, 0, mean_src=(ub_x0, ub_x1))
    spmm_pass(ub_r, ub_c, ub_v, UB_E, UB_E, UB_OUT, ub_x1, NU, 0, NBD_P,
              ub_m, NU_P, mean_src=(ub_x0, ub_x1))

    nu_pt = BATCH // NT
    perms = [iota ^ (1 << p) for p in range(4)]

    def lane_sum(s):
        for pm in perms:
            s = s + s.at[pm].get(mode="promise_in_bounds")
        return s

    @pl.loop(0, nu_pt // FB)
    def _(i):
        u_off = t * nu_pt + i * FB
        pltpu.sync_copy(users.at[pl.ds(u_off, FB)], uidx)
        pltpu.sync_copy(bundles_flat.at[pl.ds(2 * u_off, 2 * FB)],
                        bidx.at[0])

        pltpu.sync_copy(ub_m.at[h].at[uidx], ug_a)
        pltpu.sync_copy(ui_m.at[h].at[uidx], ug_b)

        @pl.loop(0, 2 * FB // L)
        def _(k):
            bd = bidx[0, pl.ds(k * L, L)]
            bidx[1, pl.ds(k * L, L)] = bd + NU_P

        pltpu.sync_copy(ub_m.at[h].at[bidx.at[1]], bg_a)
        pltpu.sync_copy(bi_o.at[h].at[bidx.at[0]], bg_b)

        @pl.loop(0, 2 * FB // L)
        def _(g):
            predv = zero16
            for e2 in range(L):
                bq = g * (L // 2) + e2 // 2
                jj = e2 % 2
                u0 = ug_a[bq, pl.ds(0, L)]
                u1 = ug_a[bq, pl.ds(L, L)]
                u2 = ug_b[bq, pl.ds(0, L)]
                u3 = ug_b[bq, pl.ds(L, L)]
                s = (u0 * bg_a[2 * bq + jj, pl.ds(0, L)]
                     + u1 * bg_a[2 * bq + jj, pl.ds(L, L)]
                     + u2 * bg_b[2 * bq + jj, pl.ds(0, L)]
                     + u3 * bg_b[2 * bq + jj, pl.ds(L, L)])
                predv = jnp.where(iota == e2, lane_sum(s), predv)
            pbuf[pl.ds(g * L, L)] = predv

        pltpu.sync_copy(pbuf, pred.at[h].at[pl.ds(2 * u_off, 2 * FB)])


@jax.jit
def _run(ub_r, ub_c, ub_v, ui_r, ui_c, ui_v, bi_r, bi_c, bi_v,
         ub_x0, ui_x0, users, bundles_flat):
    mesh = plsc.VectorSubcoreMesh(core_axis_name="c", subcore_axis_name="s")
    f32 = jnp.float32
    kfn = pl.kernel(
        _body,
        out_type=jax.ShapeDtypeStruct((2, BATCH * 2), f32),
        mesh=mesh,
        compiler_params=pltpu.CompilerParams(use_tc_tiling_on_sc=False),
        scratch_types=[
            pltpu.HBM((2, NU_P + NBD_P, DH), f32),
            pltpu.HBM((2, NU_P + NIT_P, DH), f32),
            pltpu.HBM((2, NU_P + NBD_P, DH), f32),
            pltpu.HBM((2, NU_P + NIT_P, DH), f32),
            pltpu.HBM((2, NBD_P, DH), f32),
            pltpu.VMEM_SHARED((NU_P, DH), f32),
            pltpu.VMEM((EB,), jnp.int32),
            pltpu.VMEM((EB,), jnp.int32),
            pltpu.VMEM((EB,), f32),
            pltpu.VMEM((NI, G), jnp.int32),
            pltpu.VMEM((NI, G), jnp.int32),
            pltpu.VMEM((G, DH), f32),
            pltpu.VMEM((G, DH), f32),
            pltpu.VMEM((FCH, DH), f32),
            pltpu.VMEM((FCH, DH), f32),
            pltpu.VMEM((FCH, DH), f32),
            pltpu.VMEM((FB,), jnp.int32),
            pltpu.VMEM((2, 2 * FB), jnp.int32),
            pltpu.VMEM((FB, DH), f32),
            pltpu.VMEM((FB, DH), f32),
            pltpu.VMEM((2 * FB, DH), f32),
            pltpu.VMEM((2 * FB, DH), f32),
            pltpu.VMEM((2 * FB,), f32),
            pltpu.SemaphoreType.DMA,
            pltpu.SemaphoreType.DMA,
        ],
    )
    zrows = jnp.zeros((NU_P // NT, DH), jnp.float32)
    return kfn(ub_r, ub_c, ub_v, ui_r, ui_c, ui_v, bi_r, bi_c, bi_v,
               ub_x0, ui_x0, users, bundles_flat, zrows)


def _pad(x, n):
    return jnp.concatenate([x, jnp.zeros((n - x.shape[0],), x.dtype)])


def kernel(users_feature, bundles_feature, items_feature, ub_vals, ui_vals,
           bi_vals, ub_rows, ub_cols, ui_rows, ui_cols, bi_rows, bi_cols,
           users, bundles):
    def _halves(a):
        return jnp.stack([a[:, :DH], a[:, DH:]], axis=0)

    ub_x0 = jnp.zeros((2, NU_P + NBD_P, DH), jnp.float32)
    ub_x0 = ub_x0.at[:, :NU].set(_halves(users_feature))
    ub_x0 = ub_x0.at[:, NU_P:NU_P + NBD].set(_halves(bundles_feature))
    ui_x0 = jnp.zeros((2, NU_P + NIT_P, DH), jnp.float32)
    ui_x0 = ui_x0.at[:, :NU].set(_halves(users_feature))
    ui_x0 = ui_x0.at[:, NU_P:NU_P + NIT].set(_halves(items_feature))

    pred2 = _run(
        _pad(ub_rows, UB_PAD), _pad(ub_cols, UB_PAD), _pad(ub_vals, UB_PAD),
        _pad(ui_rows, UI_PAD), _pad(ui_cols, UI_PAD), _pad(ui_vals, UI_PAD),
        _pad(bi_rows, BI_PAD), _pad(bi_cols, BI_PAD), _pad(bi_vals, BI_PAD),
        ub_x0, ui_x0, users, bundles.reshape(-1))

    p = (pred2[0] + pred2[1]).reshape(BATCH, 2)
    bpr = jnp.mean(jax.nn.softplus(p[:, 1] - p[:, 0]))
    return (bpr, jnp.zeros((1,), jnp.float32))
